# baseline (speedup 1.0000x reference)
import functools

import jax
import jax.numpy as jnp
from jax import lax
from jax.experimental import pallas as pl
from jax.experimental.pallas import tpu as pltpu
from jax.experimental.pallas import tpu_sc as plsc

NC = 2
NS = 16
NW = NC * NS


def _sc_deg_body(nch, rpt, dst3_hbm, ones_hbm, zeros_hbm, deg_hbm,
                 dstv, onesv, acc, sem):
    c = lax.axis_index("c")
    s = lax.axis_index("s")
    wid = c * NS + s
    row0 = s * rpt
    pltpu.sync_copy(zeros_hbm.at[pl.ds(row0, rpt)], acc.at[pl.ds(row0, rpt)])
    pltpu.sync_copy(ones_hbm, onesv)
    pltpu.sync_copy(dst3_hbm.at[wid], dstv)
    plsc.subcore_barrier()

    def body(j, carry):
        pltpu.sync_copy(onesv, acc.at[dstv.at[j]], add=True)
        return carry

    lax.fori_loop(0, nch, body, 0)
    plsc.subcore_barrier()
    pltpu.sync_copy(acc.at[pl.ds(row0, rpt)], deg_hbm.at[c, pl.ds(row0, rpt)])


def _sc_edge_body(nch, rpt, hs_hbm, src3_hbm, dst3_hbm, zeros_hbm, p_hbm,
                  srcv, dstv, rows, acc, sem):
    c = lax.axis_index("c")
    s = lax.axis_index("s")
    wid = c * NS + s
    row0 = s * rpt

    @pl.when(c == 0)
    def _():
        pltpu.sync_copy(hs_hbm.at[pl.ds(row0, rpt)], acc.at[pl.ds(row0, rpt)])

    @pl.when(c == 1)
    def _():
        pltpu.sync_copy(zeros_hbm.at[pl.ds(row0, rpt)],
                        acc.at[pl.ds(row0, rpt)])

    pltpu.sync_copy(src3_hbm.at[wid], srcv)
    pltpu.sync_copy(dst3_hbm.at[wid], dstv)
    plsc.subcore_barrier()

    def body(j, carry):
        pltpu.async_copy(hs_hbm.at[srcv.at[j]], rows, sem).wait()
        pltpu.sync_copy(rows, acc.at[dstv.at[j]], add=True)
        return carry

    lax.fori_loop(0, nch, body, 0)
    plsc.subcore_barrier()
    pltpu.sync_copy(acc.at[pl.ds(row0, rpt)], p_hbm.at[c, pl.ds(row0, rpt)])


def _tc_mm_body(x_ref, w_ref, h_ref):
    h_ref[...] = jnp.dot(x_ref[...], w_ref[...],
                         preferred_element_type=jnp.float32)


def _tc_scale_body(degp_ref, h_ref, hs_ref):
    dis = lax.rsqrt(degp_ref[0, :, 0:1] + degp_ref[1, :, 0:1] + 1.0)
    hs_ref[...] = dis * h_ref[...]


def _tc2_body(degp_ref, p_ref, b_ref, w_ref, hs_ref):
    dis = lax.rsqrt(degp_ref[0, :, 0:1] + degp_ref[1, :, 0:1] + 1.0)
    agg = p_ref[0] + p_ref[1]
    z = jnp.maximum(dis * agg + b_ref[...], 0.0)
    hs_ref[...] = dis * jnp.dot(z, w_ref[...],
                                preferred_element_type=jnp.float32)


def _tc3_body(degp_ref, q_ref, b_ref, out_ref):
    dis = lax.rsqrt(degp_ref[0, :, 0:1] + degp_ref[1, :, 0:1] + 1.0)
    out_ref[...] = dis * (q_ref[0] + q_ref[1]) + b_ref[...]


def kernel(x, edge_index, W1, b1, W2, b2):
    n, d_in = x.shape
    d_hid = W1.shape[1]
    d_out = W2.shape[1]
    assert d_hid == d_out
    e = edge_index.shape[1]
    assert e % NW == 0
    ept = e // NW
    chunk = 125
    assert ept % chunk == 0
    nch = ept // chunk

    npad = -(-n // (NS * 64)) * (NS * 64)
    rpt = npad // NS

    x_p = jnp.pad(x, ((0, npad - n), (0, 0)))
    src3 = edge_index[0].reshape(NW, nch, chunk)
    dst3 = edge_index[1].reshape(NW, nch, chunk)
    ones_w = jnp.ones((chunk, d_hid), jnp.float32)
    zerosd = jnp.zeros((npad, d_hid), jnp.float32)

    mesh = plsc.VectorSubcoreMesh(core_axis_name="c", subcore_axis_name="s")

    deg_p = pl.kernel(
        functools.partial(_sc_deg_body, nch, rpt),
        out_type=jax.ShapeDtypeStruct((NC, npad, d_hid), jnp.float32),
        mesh=mesh,
        scratch_types=[
            pltpu.VMEM((nch, chunk), jnp.int32),
            pltpu.VMEM((chunk, d_hid), jnp.float32),
            pltpu.VMEM_SHARED((npad, d_hid), jnp.float32),
            pltpu.SemaphoreType.DMA,
        ],
    )(dst3, ones_w, zerosd)

    edge_call = pl.kernel(
        functools.partial(_sc_edge_body, nch, rpt),
        out_type=jax.ShapeDtypeStruct((NC, npad, d_hid), jnp.float32),
        mesh=mesh,
        scratch_types=[
            pltpu.VMEM((nch, chunk), jnp.int32),
            pltpu.VMEM((nch, chunk), jnp.int32),
            pltpu.VMEM((chunk, d_hid), jnp.float32),
            pltpu.VMEM_SHARED((npad, d_hid), jnp.float32),
            pltpu.SemaphoreType.DMA,
        ],
    )

    bn = 1024
    assert npad % bn == 0
    grid = (npad // bn,)
    degp_spec = pl.BlockSpec((NC, bn, d_hid), lambda i: (0, i, 0))
    row_spec = pl.BlockSpec((bn, d_hid), lambda i: (i, 0))
    p_spec = pl.BlockSpec((NC, bn, d_hid), lambda i: (0, i, 0))
    w_spec = pl.BlockSpec((d_in, d_hid), lambda i: (0, 0))
    b_spec = pl.BlockSpec((1, d_hid), lambda i: (0, 0))

    h1 = pl.pallas_call(
        _tc_mm_body,
        grid=grid,
        in_specs=[pl.BlockSpec((bn, d_in), lambda i: (i, 0)), w_spec],
        out_specs=row_spec,
        out_shape=jax.ShapeDtypeStruct((npad, d_hid), jnp.float32),
    )(x_p, W1)

    hs1 = pl.pallas_call(
        _tc_scale_body,
        grid=grid,
        in_specs=[degp_spec, row_spec],
        out_specs=row_spec,
        out_shape=jax.ShapeDtypeStruct((npad, d_hid), jnp.float32),
    )(deg_p, h1)

    p1 = edge_call(hs1, src3, dst3, zerosd)

    hs2 = pl.pallas_call(
        _tc2_body,
        grid=grid,
        in_specs=[degp_spec, p_spec, b_spec,
                  pl.BlockSpec((d_hid, d_out), lambda i: (0, 0))],
        out_specs=pl.BlockSpec((bn, d_out), lambda i: (i, 0)),
        out_shape=jax.ShapeDtypeStruct((npad, d_out), jnp.float32),
    )(deg_p, p1, b1.reshape(1, d_hid), W2)

    p2 = edge_call(hs2, src3, dst3, zerosd)

    out = pl.pallas_call(
        _tc3_body,
        grid=grid,
        in_specs=[degp_spec,
                  pl.BlockSpec((NC, bn, d_out), lambda i: (0, i, 0)),
                  pl.BlockSpec((1, d_out), lambda i: (0, 0))],
        out_specs=pl.BlockSpec((bn, d_out), lambda i: (i, 0)),
        out_shape=jax.ShapeDtypeStruct((npad, d_out), jnp.float32),
    )(deg_p, p2, b2.reshape(1, d_out))

    return out[:n]


# speedup vs baseline: 1.0004x; 1.0004x over previous
import functools

import jax
import jax.numpy as jnp
from jax import lax
from jax.experimental import pallas as pl
from jax.experimental.pallas import tpu as pltpu
from jax.experimental.pallas import tpu_sc as plsc

NC = 2
NS = 16
NW = NC * NS


def _sc_deg_body(nch, rpt, dst3_hbm, ones_hbm, zeros_hbm, deg_hbm,
                 dstv, onesv, acc, sem):
    c = lax.axis_index("c")
    s = lax.axis_index("s")
    wid = c * NS + s
    row0 = s * rpt
    pltpu.sync_copy(zeros_hbm.at[pl.ds(row0, rpt)], acc.at[pl.ds(row0, rpt)])
    pltpu.sync_copy(ones_hbm, onesv)
    pltpu.sync_copy(dst3_hbm.at[wid], dstv)
    plsc.subcore_barrier()

    def body(j, carry):
        pltpu.sync_copy(onesv, acc.at[dstv.at[j]], add=True)
        return carry

    lax.fori_loop(0, nch, body, 0)
    plsc.subcore_barrier()
    pltpu.sync_copy(acc.at[pl.ds(row0, rpt)], deg_hbm.at[c, pl.ds(row0, rpt)])


def _sc_edge_body(nch, rpt, hs_hbm, src3_hbm, dst3_hbm, zeros_hbm, p_hbm,
                  srcv, dstv, rows, acc, sem):
    c = lax.axis_index("c")
    s = lax.axis_index("s")
    wid = c * NS + s
    row0 = s * rpt

    @pl.when(c == 0)
    def _():
        pltpu.sync_copy(hs_hbm.at[pl.ds(row0, rpt)], acc.at[pl.ds(row0, rpt)])

    @pl.when(c == 1)
    def _():
        pltpu.sync_copy(zeros_hbm.at[pl.ds(row0, rpt)],
                        acc.at[pl.ds(row0, rpt)])

    pltpu.sync_copy(src3_hbm.at[wid], srcv)
    pltpu.sync_copy(dst3_hbm.at[wid], dstv)
    plsc.subcore_barrier()

    def body(j, carry):
        pltpu.async_copy(hs_hbm.at[srcv.at[j]], rows, sem).wait()
        pltpu.sync_copy(rows, acc.at[dstv.at[j]], add=True)
        return carry

    lax.fori_loop(0, nch, body, 0)
    plsc.subcore_barrier()
    pltpu.sync_copy(acc.at[pl.ds(row0, rpt)], p_hbm.at[c, pl.ds(row0, rpt)])


def _tc1_body(degp_ref, x_ref, w_ref, hs_ref):
    dis = lax.rsqrt(degp_ref[0, :, 0:1] + degp_ref[1, :, 0:1] + 1.0)
    h = jnp.dot(x_ref[...], w_ref[...], preferred_element_type=jnp.float32)
    hs_ref[...] = dis * h


def _tc2_body(degp_ref, p_ref, b_ref, w_ref, hs_ref):
    dis = lax.rsqrt(degp_ref[0, :, 0:1] + degp_ref[1, :, 0:1] + 1.0)
    agg = p_ref[0] + p_ref[1]
    z = jnp.maximum(dis * agg + b_ref[...], 0.0)
    hs_ref[...] = dis * jnp.dot(z, w_ref[...],
                                preferred_element_type=jnp.float32)


def _tc3_body(degp_ref, q_ref, b_ref, out_ref):
    dis = lax.rsqrt(degp_ref[0, :, 0:1] + degp_ref[1, :, 0:1] + 1.0)
    out_ref[...] = dis * (q_ref[0] + q_ref[1]) + b_ref[...]


def kernel(x, edge_index, W1, b1, W2, b2):
    n, d_in = x.shape
    d_hid = W1.shape[1]
    d_out = W2.shape[1]
    assert d_hid == d_out
    e = edge_index.shape[1]
    assert e % NW == 0
    ept = e // NW
    chunk = 125
    assert ept % chunk == 0
    nch = ept // chunk

    npad = -(-n // (NS * 64)) * (NS * 64)
    rpt = npad // NS

    x_p = jnp.pad(x, ((0, npad - n), (0, 0)))
    src3 = edge_index[0].reshape(NW, nch, chunk)
    dst3 = edge_index[1].reshape(NW, nch, chunk)
    ones_w = jnp.ones((chunk, d_hid), jnp.float32)
    zerosd = jnp.zeros((npad, d_hid), jnp.float32)

    mesh = plsc.VectorSubcoreMesh(core_axis_name="c", subcore_axis_name="s")

    deg_p = pl.kernel(
        functools.partial(_sc_deg_body, nch, rpt),
        out_type=jax.ShapeDtypeStruct((NC, npad, d_hid), jnp.float32),
        mesh=mesh,
        scratch_types=[
            pltpu.VMEM((nch, chunk), jnp.int32),
            pltpu.VMEM((chunk, d_hid), jnp.float32),
            pltpu.VMEM_SHARED((npad, d_hid), jnp.float32),
            pltpu.SemaphoreType.DMA,
        ],
    )(dst3, ones_w, zerosd)

    edge_call = pl.kernel(
        functools.partial(_sc_edge_body, nch, rpt),
        out_type=jax.ShapeDtypeStruct((NC, npad, d_hid), jnp.float32),
        mesh=mesh,
        scratch_types=[
            pltpu.VMEM((nch, chunk), jnp.int32),
            pltpu.VMEM((nch, chunk), jnp.int32),
            pltpu.VMEM((chunk, d_hid), jnp.float32),
            pltpu.VMEM_SHARED((npad, d_hid), jnp.float32),
            pltpu.SemaphoreType.DMA,
        ],
    )

    bn = 1024
    assert npad % bn == 0
    grid = (npad // bn,)
    degp_spec = pl.BlockSpec((NC, bn, d_hid), lambda i: (0, i, 0))
    row_spec = pl.BlockSpec((bn, d_hid), lambda i: (i, 0))
    p_spec = pl.BlockSpec((NC, bn, d_hid), lambda i: (0, i, 0))
    w_spec = pl.BlockSpec((d_in, d_hid), lambda i: (0, 0))
    b_spec = pl.BlockSpec((1, d_hid), lambda i: (0, 0))

    hs1 = pl.pallas_call(
        _tc1_body,
        grid=grid,
        in_specs=[degp_spec, pl.BlockSpec((bn, d_in), lambda i: (i, 0)),
                  w_spec],
        out_specs=row_spec,
        out_shape=jax.ShapeDtypeStruct((npad, d_hid), jnp.float32),
    )(deg_p, x_p, W1)

    p1 = edge_call(hs1, src3, dst3, zerosd)

    hs2 = pl.pallas_call(
        _tc2_body,
        grid=grid,
        in_specs=[degp_spec, p_spec, b_spec,
                  pl.BlockSpec((d_hid, d_out), lambda i: (0, 0))],
        out_specs=pl.BlockSpec((bn, d_out), lambda i: (i, 0)),
        out_shape=jax.ShapeDtypeStruct((npad, d_out), jnp.float32),
    )(deg_p, p1, b1.reshape(1, d_hid), W2)

    p2 = edge_call(hs2, src3, dst3, zerosd)

    out = pl.pallas_call(
        _tc3_body,
        grid=grid,
        in_specs=[degp_spec,
                  pl.BlockSpec((NC, bn, d_out), lambda i: (0, i, 0)),
                  pl.BlockSpec((1, d_out), lambda i: (0, 0))],
        out_specs=pl.BlockSpec((bn, d_out), lambda i: (i, 0)),
        out_shape=jax.ShapeDtypeStruct((npad, d_out), jnp.float32),
    )(deg_p, p2, b2.reshape(1, d_out))

    return out[:n]
